# baseline (speedup 1.0000x reference)
import jax
import jax.numpy as jnp
from jax.experimental import pallas as pl
from jax.experimental.pallas import tpu as pltpu


def _make_mse_kernel(num_blocks, inv_count):
    def _mse_mean_kernel(g1_ref, g2_ref, g1h_ref, g2h_ref, out_ref):
        i = pl.program_id(0)
        d1 = g1_ref[...] - g1h_ref[...]
        d2 = g2_ref[...] - g2h_ref[...]
        partial = jnp.sum(d1 * d1) + jnp.sum(d2 * d2)

        @pl.when(i == 0)
        def _init():
            out_ref[0] = 0.0

        out_ref[0] += partial

        @pl.when(i == num_blocks - 1)
        def _finish():
            out_ref[0] = out_ref[0] * inv_count

    return _mse_mean_kernel


def kernel(x, target, g1, g2, g1_hat, g2_hat):
    N, D = g1.shape
    BLOCK = 256
    grid = (N // BLOCK,)
    spec = pl.BlockSpec((BLOCK, D), lambda i: (i, 0))
    total = pl.pallas_call(
        _make_mse_kernel(N // BLOCK, 1.0 / (N * D)),
        grid=grid,
        in_specs=[spec, spec, spec, spec],
        out_specs=pl.BlockSpec(memory_space=pltpu.SMEM),
        out_shape=jax.ShapeDtypeStruct((1,), jnp.float32),
    )(g1, g2, g1_hat, g2_hat)
    return total[0]


# speedup vs baseline: 1.1179x; 1.1179x over previous
import jax
import jax.numpy as jnp
from jax.experimental import pallas as pl
from jax.experimental.pallas import tpu as pltpu


def _make_mse_kernel(num_blocks, inv_count):
    def _mse_mean_kernel(g1_ref, g2_ref, g1h_ref, g2h_ref, out_ref):
        i = pl.program_id(0)
        d1 = g1_ref[...] - g1h_ref[...]
        d2 = g2_ref[...] - g2h_ref[...]
        partial = jnp.sum(d1 * d1) + jnp.sum(d2 * d2)

        @pl.when(i == 0)
        def _init():
            out_ref[0] = 0.0

        out_ref[0] += partial

        @pl.when(i == num_blocks - 1)
        def _finish():
            out_ref[0] = out_ref[0] * inv_count

    return _mse_mean_kernel


def kernel(x, target, g1, g2, g1_hat, g2_hat):
    N, D = g1.shape
    BLOCK = 512
    grid = (N // BLOCK,)
    spec = pl.BlockSpec((BLOCK, D), lambda i: (i, 0))
    total = pl.pallas_call(
        _make_mse_kernel(N // BLOCK, 1.0 / (N * D)),
        grid=grid,
        in_specs=[spec, spec, spec, spec],
        out_specs=pl.BlockSpec(memory_space=pltpu.SMEM),
        out_shape=jax.ShapeDtypeStruct((1,), jnp.float32),
    )(g1, g2, g1_hat, g2_hat)
    return total[0]
